# baseline (speedup 1.0000x reference)
import functools

import jax
import jax.numpy as jnp
from jax.experimental import pallas as pl
from jax.experimental.pallas import tpu as pltpu

_RNN_H = 64
_CT = 128

_TA = (((0,), (0,)), ((), ()))


def _dott(w, x):
    return jax.lax.dot_general(w, x, dimension_numbers=_TA,
                               preferred_element_type=jnp.float32)


def _fused_kernel(s_ref, a_ref, r_ref, d_ref, slab_ref, hn_ref, ri_ref,
                  lv_ref, h_scr, y_scr, *, ct):
    tc = pl.program_id(0)

    @pl.when(tc == 0)
    def _():
        h_scr[...] = jnp.zeros_like(h_scr)

    B = s_ref.shape[2]
    dobs = s_ref.shape[0]
    ones1 = jnp.ones((1, B), jnp.float32)

    wx = slab_ref[0:dobs, 0:_RNN_H]
    ward = jnp.concatenate(
        [slab_ref[dobs:dobs + 3, 0:_RNN_H],
         slab_ref[512:513, 0:_RNN_H]], axis=0)
    wh = slab_ref[128:128 + _RNN_H, 0:_RNN_H]
    w1a = jnp.concatenate(
        [slab_ref[256:256 + _RNN_H, :], slab_ref[513:514, :]], axis=0)
    whda = jnp.concatenate(
        [slab_ref[384:512, 64:128], slab_ref[514:515, 64:128]], axis=0)

    def y_slot(i):
        return ((i % 4) * 128, ((i % 4) + 1) * 128)

    def heads_rv(i):
        lo, hi = y_slot(i)
        yaug = jnp.concatenate([y_scr[lo:hi, :], ones1], axis=0)
        rv = _dott(whda, yaug)
        ri_ref[i:i + 1, :] = rv[0:1, :]
        lv_ref[i:i + 1, :] = rv[1:2, :]

    h = h_scr[...]
    haug = jnp.concatenate([h, ones1], axis=0)
    for i in range(ct):
        if i >= 1:
            lo, hi = y_slot(i - 1)
            y_scr[lo:hi, :] = jnp.maximum(_dott(w1a, haug), 0.0)
        if i >= 2:
            heads_rv(i - 2)
        ard = jnp.concatenate(
            [a_ref[i:i + 1, :], r_ref[i:i + 1, :], d_ref[i:i + 1, :], ones1],
            axis=0)
        px = _dott(wx, s_ref[:, i, :]) + _dott(ward, ard)
        h = jnp.tanh(px + _dott(wh, h))
        hn_ref[i] = h
        haug = jnp.concatenate([h, ones1], axis=0)
    h_scr[...] = h

    lo, hi = y_slot(ct - 1)
    y_scr[lo:hi, :] = jnp.maximum(_dott(w1a, haug), 0.0)
    for i in range(max(ct - 2, 0), ct):
        heads_rv(i)


def kernel(slab, s, a, r, d):
    s = jnp.asarray(s, jnp.float32)
    a = jnp.asarray(a, jnp.float32)
    r = jnp.asarray(r, jnp.float32)
    d = jnp.asarray(d, jnp.float32)
    T, B, Dobs = s.shape
    ct = _CT if T % _CT == 0 else 1

    st = jnp.transpose(s, (2, 0, 1))

    hn_t, ri, lv = pl.pallas_call(
        functools.partial(_fused_kernel, ct=ct),
        out_shape=(jax.ShapeDtypeStruct((T, _RNN_H, B), jnp.float32),
                   jax.ShapeDtypeStruct((T, B), jnp.float32),
                   jax.ShapeDtypeStruct((T, B), jnp.float32)),
        grid=(T // ct,),
        in_specs=[
            pl.BlockSpec((Dobs, ct, B), lambda t: (0, t, 0)),
            pl.BlockSpec((ct, B), lambda t: (t, 0)),
            pl.BlockSpec((ct, B), lambda t: (t, 0)),
            pl.BlockSpec((ct, B), lambda t: (t, 0)),
            pl.BlockSpec((520, 128), lambda t: (0, 0)),
        ],
        out_specs=(pl.BlockSpec((ct, _RNN_H, B), lambda t: (t, 0, 0)),
                   pl.BlockSpec((ct, B), lambda t: (t, 0)),
                   pl.BlockSpec((ct, B), lambda t: (t, 0))),
        scratch_shapes=[pltpu.VMEM((_RNN_H, B), jnp.float32),
                        pltpu.VMEM((4 * 128, B), jnp.float32)],
        compiler_params=pltpu.CompilerParams(
            dimension_semantics=("arbitrary",)),
    )(st, a, r, d, slab)

    hn = jnp.transpose(hn_t, (0, 2, 1))
    return ri[..., None], lv[..., None], hn


# speedup vs baseline: 1.0184x; 1.0184x over previous
import functools

import jax
import jax.numpy as jnp
from jax.experimental import pallas as pl
from jax.experimental.pallas import tpu as pltpu

_RNN_H = 64
_CT = 64

_TA = (((0,), (0,)), ((), ()))


def _dott(w, x):
    return jax.lax.dot_general(w, x, dimension_numbers=_TA,
                               preferred_element_type=jnp.float32)


def _fused_kernel(s_ref, a_ref, r_ref, d_ref, slab_ref, hn_ref, ri_ref,
                  lv_ref, h_scr, y_scr, *, ct):
    tc = pl.program_id(0)

    @pl.when(tc == 0)
    def _():
        h_scr[...] = jnp.zeros_like(h_scr)

    B = s_ref.shape[2]
    dobs = s_ref.shape[0]
    ones1 = jnp.ones((1, B), jnp.float32)

    wx = slab_ref[0:dobs, 0:_RNN_H]
    ward = jnp.concatenate(
        [slab_ref[dobs:dobs + 3, 0:_RNN_H],
         slab_ref[512:513, 0:_RNN_H]], axis=0)
    wh = slab_ref[128:128 + _RNN_H, 0:_RNN_H]
    w1a = jnp.concatenate(
        [slab_ref[256:256 + _RNN_H, :], slab_ref[513:514, :]], axis=0)
    whda = jnp.concatenate(
        [slab_ref[384:512, 64:128], slab_ref[514:515, 64:128]], axis=0)

    def y_slot(i):
        return ((i % 4) * 128, ((i % 4) + 1) * 128)

    def heads_rv(i):
        lo, hi = y_slot(i)
        yaug = jnp.concatenate([y_scr[lo:hi, :], ones1], axis=0)
        rv = _dott(whda, yaug)
        ri_ref[i:i + 1, :] = rv[0:1, :]
        lv_ref[i:i + 1, :] = rv[1:2, :]

    h = h_scr[...]
    haug = jnp.concatenate([h, ones1], axis=0)
    for i in range(ct):
        if i >= 1:
            lo, hi = y_slot(i - 1)
            y_scr[lo:hi, :] = jnp.maximum(_dott(w1a, haug), 0.0)
        if i >= 2:
            heads_rv(i - 2)
        ard = jnp.concatenate(
            [a_ref[i:i + 1, :], r_ref[i:i + 1, :], d_ref[i:i + 1, :], ones1],
            axis=0)
        px = _dott(wx, s_ref[:, i, :]) + _dott(ward, ard)
        h = jnp.tanh(px + _dott(wh, h))
        hn_ref[i] = h
        haug = jnp.concatenate([h, ones1], axis=0)
    h_scr[...] = h

    lo, hi = y_slot(ct - 1)
    y_scr[lo:hi, :] = jnp.maximum(_dott(w1a, haug), 0.0)
    for i in range(max(ct - 2, 0), ct):
        heads_rv(i)


def kernel(slab, s, a, r, d):
    s = jnp.asarray(s, jnp.float32)
    a = jnp.asarray(a, jnp.float32)
    r = jnp.asarray(r, jnp.float32)
    d = jnp.asarray(d, jnp.float32)
    T, B, Dobs = s.shape
    ct = _CT if T % _CT == 0 else 1

    st = jnp.transpose(s, (2, 0, 1))

    hn_t, ri, lv = pl.pallas_call(
        functools.partial(_fused_kernel, ct=ct),
        out_shape=(jax.ShapeDtypeStruct((T, _RNN_H, B), jnp.float32),
                   jax.ShapeDtypeStruct((T, B), jnp.float32),
                   jax.ShapeDtypeStruct((T, B), jnp.float32)),
        grid=(T // ct,),
        in_specs=[
            pl.BlockSpec((Dobs, ct, B), lambda t: (0, t, 0)),
            pl.BlockSpec((ct, B), lambda t: (t, 0)),
            pl.BlockSpec((ct, B), lambda t: (t, 0)),
            pl.BlockSpec((ct, B), lambda t: (t, 0)),
            pl.BlockSpec((520, 128), lambda t: (0, 0)),
        ],
        out_specs=(pl.BlockSpec((ct, _RNN_H, B), lambda t: (t, 0, 0)),
                   pl.BlockSpec((ct, B), lambda t: (t, 0)),
                   pl.BlockSpec((ct, B), lambda t: (t, 0))),
        scratch_shapes=[pltpu.VMEM((_RNN_H, B), jnp.float32),
                        pltpu.VMEM((4 * 128, B), jnp.float32)],
        compiler_params=pltpu.CompilerParams(
            dimension_semantics=("arbitrary",)),
    )(st, a, r, d, slab)

    hn = jnp.transpose(hn_t, (0, 2, 1))
    return ri[..., None], lv[..., None], hn
